# Initial kernel scaffold; baseline (speedup 1.0000x reference)
import functools

import jax
import jax.numpy as jnp
from jax import lax
from jax.experimental import pallas as pl
from jax.experimental.pallas import tpu as pltpu
from jax.experimental.pallas import tpu_sc as plsc

N = 10000
E = 320000
D_IN = 128
D_EDGE = 16
BOT = 8
HID = 256
G = 64

NC = 2
NS = 16
LANES = 16
CH = HID // NC
CHUNK = 128
NCHUNK = E // CHUNK
ROWS_PER_SUB = N // NS
SC_ITERS = (NCHUNK + NS - 1) // NS

BLK_E = 8000
BLK_N = 2500


def _prep_body(x_ref, gates_ref, win_ref, bin_ref,
               ws1_ref, bs1_ref, wm1_ref, bm1_ref, be1_ref, wux1_ref, bux1_ref,
               ws2_ref, bs2_ref, wm2_ref, bm2_ref, be2_ref,
               z1p_ref, old1_ref, a2_ref, c2_ref):
    h = x_ref[...] * jax.nn.sigmoid(gates_ref[...])
    h = jax.nn.relu(jnp.dot(h, win_ref[...].T,
                            preferred_element_type=jnp.float32) + bin_ref[...])
    wm1a = wm1_ref[:, :HID]
    wm1b = wm1_ref[:, HID:]
    a1 = jnp.dot(wm1a, ws1_ref[...], preferred_element_type=jnp.float32)
    c1 = (jnp.dot(bs1_ref[...], wm1a.T, preferred_element_type=jnp.float32)
          + jnp.dot(be1_ref[...], wm1b.T, preferred_element_type=jnp.float32)
          + bm1_ref[...])
    z1 = jnp.dot(h, a1.T, preferred_element_type=jnp.float32) + c1
    z1p_ref[0] = z1[:, :CH]
    z1p_ref[1] = z1[:, CH:]
    old1_ref[...] = jnp.dot(h, wux1_ref[...].T,
                            preferred_element_type=jnp.float32) + bux1_ref[...]
    wm2a = wm2_ref[:, :HID]
    wm2b = wm2_ref[:, HID:]
    a2_ref[...] = jnp.dot(wm2a, ws2_ref[...], preferred_element_type=jnp.float32)
    c2_ref[...] = (jnp.dot(bs2_ref[...], wm2a.T, preferred_element_type=jnp.float32)
                   + jnp.dot(be2_ref[...], wm2b.T, preferred_element_type=jnp.float32)
                   + bm2_ref[...])


def _prep(x, gates, win, bin_, c1p, c2p):
    return pl.pallas_call(
        _prep_body,
        out_shape=(
            jax.ShapeDtypeStruct((NC, N, CH), jnp.float32),
            jax.ShapeDtypeStruct((N, HID), jnp.float32),
            jax.ShapeDtypeStruct((HID, HID), jnp.float32),
            jax.ShapeDtypeStruct((1, HID), jnp.float32),
        ),
    )(x, gates, win, bin_,
      c1p['Ws'], c1p['bs'].reshape(1, HID), c1p['Wm'],
      c1p['bm'].reshape(1, HID), c1p['be'].reshape(1, HID),
      c1p['Wux'], c1p['bux'].reshape(1, HID),
      c2p['Ws'], c2p['bs'].reshape(1, HID), c2p['Wm'],
      c2p['bm'].reshape(1, HID), c2p['be'].reshape(1, HID))


def _edges_body(ea_ref, wm1_ref, we1_ref, wm2_ref, we2_ref, e1p_ref, e2p_ref):
    b1 = jnp.dot(wm1_ref[:, HID:], we1_ref[...],
                 preferred_element_type=jnp.float32)
    b2 = jnp.dot(wm2_ref[:, HID:], we2_ref[...],
                 preferred_element_type=jnp.float32)
    ea = ea_ref[...]
    e1 = jnp.dot(ea, b1.T, preferred_element_type=jnp.float32)
    e2 = jnp.dot(ea, b2.T, preferred_element_type=jnp.float32)
    e1p_ref[0] = e1[:, :CH]
    e1p_ref[1] = e1[:, CH:]
    e2p_ref[0] = e2[:, :CH]
    e2p_ref[1] = e2[:, CH:]


def _edges(edge_attr, c1p, c2p):
    nblk = E // BLK_E
    return pl.pallas_call(
        _edges_body,
        grid=(nblk,),
        in_specs=[
            pl.BlockSpec((BLK_E, D_EDGE), lambda i: (i, 0)),
            pl.BlockSpec((HID, 2 * HID), lambda i: (0, 0)),
            pl.BlockSpec((HID, D_EDGE), lambda i: (0, 0)),
            pl.BlockSpec((HID, 2 * HID), lambda i: (0, 0)),
            pl.BlockSpec((HID, D_EDGE), lambda i: (0, 0)),
        ],
        out_specs=(
            pl.BlockSpec((NC, BLK_E, CH), lambda i: (0, i, 0)),
            pl.BlockSpec((NC, BLK_E, CH), lambda i: (0, i, 0)),
        ),
        out_shape=(
            jax.ShapeDtypeStruct((NC, E, CH), jnp.float32),
            jax.ShapeDtypeStruct((NC, E, CH), jnp.float32),
        ),
    )(edge_attr, c1p['Wm'], c1p['We'], c2p['Wm'], c2p['We'])


def _sc_conv_body(z_hbm, e_hbm, src_hbm, dst_hbm, out_hbm,
                  srcv, dstv, gv, ev, mv, aggr, sem):
    c = lax.axis_index("c")
    s = lax.axis_index("s")

    @pl.loop(0, CHUNK)
    def _zero(i):
        for j in range(CH // LANES):
            mv.at[i, pl.ds(j * LANES, LANES)][...] = jnp.zeros(
                (LANES,), jnp.float32)

    base_row = s * ROWS_PER_SUB
    for q in range(ROWS_PER_SUB // CHUNK):
        pltpu.sync_copy(mv, aggr.at[pl.ds(base_row + q * CHUNK, CHUNK)])
    rem = ROWS_PER_SUB % CHUNK
    if rem:
        pltpu.sync_copy(mv.at[pl.ds(0, rem)],
                        aggr.at[pl.ds(base_row + ROWS_PER_SUB - rem, rem)])
    plsc.subcore_barrier()

    @pl.loop(0, SC_ITERS)
    def _chunk(it):
        k_id = it * NS + s

        @pl.when(k_id < NCHUNK)
        def _():
            base = k_id * CHUNK
            pltpu.sync_copy(src_hbm.at[pl.ds(base, CHUNK)], srcv)
            pltpu.sync_copy(dst_hbm.at[pl.ds(base, CHUNK)], dstv)
            pltpu.async_copy(z_hbm.at[c].at[srcv], gv, sem).wait()
            pltpu.sync_copy(e_hbm.at[c].at[pl.ds(base, CHUNK)], ev)

            @pl.loop(0, CHUNK)
            def _edge(i):
                for j in range(CH // LANES):
                    sl = pl.ds(j * LANES, LANES)
                    mv.at[i, sl][...] = jnp.maximum(
                        gv.at[i, sl][...] + ev.at[i, sl][...], 0.0)

            pltpu.sync_copy(mv, aggr.at[dstv], add=True)

    plsc.subcore_barrier()
    for q in range(ROWS_PER_SUB // CHUNK):
        row = base_row + q * CHUNK
        pltpu.sync_copy(aggr.at[pl.ds(row, CHUNK)],
                        out_hbm.at[c].at[pl.ds(row, CHUNK)])
    if rem:
        row = base_row + ROWS_PER_SUB - rem
        pltpu.sync_copy(aggr.at[pl.ds(row, rem)],
                        out_hbm.at[c].at[pl.ds(row, rem)])


def _sc_conv(zp, ep, src, dst):
    mesh = plsc.VectorSubcoreMesh(core_axis_name="c", subcore_axis_name="s")
    return pl.kernel(
        _sc_conv_body,
        out_type=jax.ShapeDtypeStruct((NC, N, CH), jnp.float32),
        mesh=mesh,
        scratch_types=[
            pltpu.VMEM((CHUNK,), jnp.int32),
            pltpu.VMEM((CHUNK,), jnp.int32),
            pltpu.VMEM((CHUNK, CH), jnp.float32),
            pltpu.VMEM((CHUNK, CH), jnp.float32),
            pltpu.VMEM((CHUNK, CH), jnp.float32),
            pltpu.VMEM_SHARED((N, CH), jnp.float32),
            pltpu.SemaphoreType.DMA,
        ],
    )(zp, ep, src, dst)


def _update_body(old_ref, aggr_ref, wg_ref, bg_ref, wup_ref, bup_ref,
                 hpre_ref, stats_ref):
    i = pl.program_id(0)
    old = old_ref[...]
    cat = jnp.concatenate([old, aggr_ref[0], aggr_ref[1]], axis=-1)
    gate = jax.nn.sigmoid(jnp.dot(cat, wg_ref[...].T,
                                  preferred_element_type=jnp.float32)
                          + bg_ref[...])
    upd = jax.nn.relu(jnp.dot(cat, wup_ref[...].T,
                              preferred_element_type=jnp.float32)
                      + bup_ref[...])
    h = gate * upd + (1.0 - gate) * old
    hpre_ref[...] = h

    @pl.when(i == 0)
    def _():
        stats_ref[...] = jnp.zeros_like(stats_ref)

    stats_ref[0:1, :] += jnp.sum(h, axis=0, keepdims=True)
    stats_ref[1:2, :] += jnp.sum(h * h, axis=0, keepdims=True)


def _update(old, aggrp, cp):
    nblk = N // BLK_N
    return pl.pallas_call(
        _update_body,
        grid=(nblk,),
        in_specs=[
            pl.BlockSpec((BLK_N, HID), lambda i: (i, 0)),
            pl.BlockSpec((NC, BLK_N, CH), lambda i: (0, i, 0)),
            pl.BlockSpec((HID, 2 * HID), lambda i: (0, 0)),
            pl.BlockSpec((1, HID), lambda i: (0, 0)),
            pl.BlockSpec((HID, 2 * HID), lambda i: (0, 0)),
            pl.BlockSpec((1, HID), lambda i: (0, 0)),
        ],
        out_specs=(
            pl.BlockSpec((BLK_N, HID), lambda i: (i, 0)),
            pl.BlockSpec((8, HID), lambda i: (0, 0)),
        ),
        out_shape=(
            jax.ShapeDtypeStruct((N, HID), jnp.float32),
            jax.ShapeDtypeStruct((8, HID), jnp.float32),
        ),
    )(old, aggrp, cp['Wg'], cp['bg'].reshape(1, HID),
      cp['Wup'], cp['bup'].reshape(1, HID))


def _bn_next_body(hpre_ref, stats_ref, gamma_ref, beta_ref,
                  a2_ref, c2_ref, wux_ref, bux_ref,
                  z2p_ref, old2_ref):
    mean = stats_ref[0:1, :] / N
    var = stats_ref[1:2, :] / N - mean * mean
    h = jax.nn.relu(gamma_ref[...] * (hpre_ref[...] - mean)
                    * jax.lax.rsqrt(var + 1e-5) + beta_ref[...])
    z2 = jnp.dot(h, a2_ref[...].T, preferred_element_type=jnp.float32) \
        + c2_ref[...]
    z2p_ref[0] = z2[:, :CH]
    z2p_ref[1] = z2[:, CH:]
    old2_ref[...] = jnp.dot(h, wux_ref[...].T,
                            preferred_element_type=jnp.float32) + bux_ref[...]


def _bn_next(hpre, stats, gamma, beta, a2, c2, c2p):
    nblk = N // BLK_N
    return pl.pallas_call(
        _bn_next_body,
        grid=(nblk,),
        in_specs=[
            pl.BlockSpec((BLK_N, HID), lambda i: (i, 0)),
            pl.BlockSpec((8, HID), lambda i: (0, 0)),
            pl.BlockSpec((1, HID), lambda i: (0, 0)),
            pl.BlockSpec((1, HID), lambda i: (0, 0)),
            pl.BlockSpec((HID, HID), lambda i: (0, 0)),
            pl.BlockSpec((1, HID), lambda i: (0, 0)),
            pl.BlockSpec((HID, HID), lambda i: (0, 0)),
            pl.BlockSpec((1, HID), lambda i: (0, 0)),
        ],
        out_specs=(
            pl.BlockSpec((NC, BLK_N, CH), lambda i: (0, i, 0)),
            pl.BlockSpec((BLK_N, HID), lambda i: (i, 0)),
        ),
        out_shape=(
            jax.ShapeDtypeStruct((NC, N, CH), jnp.float32),
            jax.ShapeDtypeStruct((N, HID), jnp.float32),
        ),
    )(hpre, stats, gamma.reshape(1, HID), beta.reshape(1, HID),
      a2, c2, c2p['Wux'], c2p['bux'].reshape(1, HID))


def _bn_body(hpre_ref, stats_ref, gamma_ref, beta_ref, h_ref):
    mean = stats_ref[0:1, :] / N
    var = stats_ref[1:2, :] / N - mean * mean
    h_ref[...] = jax.nn.relu(gamma_ref[...] * (hpre_ref[...] - mean)
                             * jax.lax.rsqrt(var + 1e-5) + beta_ref[...])


def _bn(hpre, stats, gamma, beta):
    nblk = N // BLK_N
    return pl.pallas_call(
        _bn_body,
        grid=(nblk,),
        in_specs=[
            pl.BlockSpec((BLK_N, HID), lambda i: (i, 0)),
            pl.BlockSpec((8, HID), lambda i: (0, 0)),
            pl.BlockSpec((1, HID), lambda i: (0, 0)),
            pl.BlockSpec((1, HID), lambda i: (0, 0)),
        ],
        out_specs=pl.BlockSpec((BLK_N, HID), lambda i: (i, 0)),
        out_shape=jax.ShapeDtypeStruct((N, HID), jnp.float32),
    )(hpre, stats, gamma.reshape(1, HID), beta.reshape(1, HID))


def _head_body(h_ref, batch_ref, wfc_ref, bfc_ref,
               wreg_ref, breg_ref, wcls_ref, bcls_ref,
               reg_ref, cls_ref):
    gid = lax.broadcasted_iota(jnp.int32, (G, N), 0)
    onehot = (batch_ref[...] == gid).astype(jnp.float32)
    sums = jnp.dot(onehot, h_ref[...], preferred_element_type=jnp.float32)
    counts = jnp.sum(onehot, axis=1, keepdims=True)
    gf = sums / jnp.maximum(counts, 1.0)
    gf = jax.nn.relu(jnp.dot(gf, wfc_ref[...].T,
                             preferred_element_type=jnp.float32) + bfc_ref[...])
    reg_ref[...] = jnp.dot(gf, wreg_ref[...].T,
                           preferred_element_type=jnp.float32) + breg_ref[...]
    cls_ref[...] = jax.nn.sigmoid(
        jnp.dot(gf, wcls_ref[...].T, preferred_element_type=jnp.float32)
        + bcls_ref[...])


def _head(h, batch, params):
    return pl.pallas_call(
        _head_body,
        out_shape=(
            jax.ShapeDtypeStruct((G, 1), jnp.float32),
            jax.ShapeDtypeStruct((G, 1), jnp.float32),
        ),
    )(h, batch.reshape(1, N), params['Wfc'],
      params['bfc'].reshape(1, HID // 2),
      params['Wreg'], params['breg'].reshape(1, 1),
      params['Wcls'], params['bcls'].reshape(1, 1))


def kernel(x, edge_index, edge_attr, batch, params):
    src = edge_index[0]
    dst = edge_index[1]
    c1p = params['conv1']
    c2p = params['conv2']

    z1p, old1, a2, c2 = _prep(x, params['feature_gates'].reshape(1, D_IN),
                              params['Win'], params['bin'].reshape(1, BOT),
                              c1p, c2p)
    e1p, e2p = _edges(edge_attr, c1p, c2p)

    aggr1p = _sc_conv(z1p, e1p, src, dst)
    hpre1, stats1 = _update(old1, aggr1p, c1p)
    z2p, old2 = _bn_next(hpre1, stats1, params['g1'], params['b1'],
                         a2, c2, c2p)

    aggr2p = _sc_conv(z2p, e2p, src, dst)
    hpre2, stats2 = _update(old2, aggr2p, c2p)
    h2 = _bn(hpre2, stats2, params['g2'], params['b2'])

    reg, cls = _head(h2, batch, params)
    return (reg[:, 0], cls[:, 0])


# speedup vs baseline: 2.3016x; 2.3016x over previous
import functools

import jax
import jax.numpy as jnp
from jax import lax
from jax.experimental import pallas as pl
from jax.experimental.pallas import tpu as pltpu
from jax.experimental.pallas import tpu_sc as plsc

N = 10000
E = 320000
D_IN = 128
D_EDGE = 16
BOT = 8
HID = 256
G = 64

NC = 2
NS = 16
LANES = 16
CH = HID // NC
CHUNK = 128
NCHUNK = E // CHUNK
NROWBLK = N // CHUNK
NROWTAIL = N - NROWBLK * CHUNK
ROW_ITERS = (NROWBLK + NS - 1) // NS
SC_ITERS = (NCHUNK + NS - 1) // NS

BLK_E = 8000
BLK_N = 2000


def _prep_body(x_ref, gates_ref, win_ref, bin_ref,
               ws1_ref, bs1_ref, wm1_ref, bm1_ref, be1_ref, wux1_ref, bux1_ref,
               ws2_ref, bs2_ref, wm2_ref, bm2_ref, be2_ref,
               z1p_ref, old1_ref, a2_ref, c2_ref):
    h = x_ref[...] * jax.nn.sigmoid(gates_ref[...])
    h = jax.nn.relu(jnp.dot(h, win_ref[...].T,
                            preferred_element_type=jnp.float32) + bin_ref[...])
    wm1a = wm1_ref[:, :HID]
    wm1b = wm1_ref[:, HID:]
    a1 = jnp.dot(wm1a, ws1_ref[...], preferred_element_type=jnp.float32)
    c1 = (jnp.dot(bs1_ref[...], wm1a.T, preferred_element_type=jnp.float32)
          + jnp.dot(be1_ref[...], wm1b.T, preferred_element_type=jnp.float32)
          + bm1_ref[...])
    z1 = jnp.dot(h, a1.T, preferred_element_type=jnp.float32) + c1
    z1p_ref[0] = z1[:, :CH]
    z1p_ref[1] = z1[:, CH:]
    old1_ref[...] = jnp.dot(h, wux1_ref[...].T,
                            preferred_element_type=jnp.float32) + bux1_ref[...]
    wm2a = wm2_ref[:, :HID]
    wm2b = wm2_ref[:, HID:]
    a2_ref[...] = jnp.dot(wm2a, ws2_ref[...], preferred_element_type=jnp.float32)
    c2_ref[...] = (jnp.dot(bs2_ref[...], wm2a.T, preferred_element_type=jnp.float32)
                   + jnp.dot(be2_ref[...], wm2b.T, preferred_element_type=jnp.float32)
                   + bm2_ref[...])


def _prep(x, gates, win, bin_, c1p, c2p):
    return pl.pallas_call(
        _prep_body,
        out_shape=(
            jax.ShapeDtypeStruct((NC, N, CH), jnp.float32),
            jax.ShapeDtypeStruct((N, HID), jnp.float32),
            jax.ShapeDtypeStruct((HID, HID), jnp.float32),
            jax.ShapeDtypeStruct((1, HID), jnp.float32),
        ),
    )(x, gates, win, bin_,
      c1p['Ws'], c1p['bs'].reshape(1, HID), c1p['Wm'],
      c1p['bm'].reshape(1, HID), c1p['be'].reshape(1, HID),
      c1p['Wux'], c1p['bux'].reshape(1, HID),
      c2p['Ws'], c2p['bs'].reshape(1, HID), c2p['Wm'],
      c2p['bm'].reshape(1, HID), c2p['be'].reshape(1, HID))


def _edges_body(ea_ref, wm1_ref, we1_ref, wm2_ref, we2_ref, e1p_ref, e2p_ref):
    b1 = jnp.dot(wm1_ref[:, HID:], we1_ref[...],
                 preferred_element_type=jnp.float32)
    b2 = jnp.dot(wm2_ref[:, HID:], we2_ref[...],
                 preferred_element_type=jnp.float32)
    ea = ea_ref[...]
    e1 = jnp.dot(ea, b1.T, preferred_element_type=jnp.float32)
    e2 = jnp.dot(ea, b2.T, preferred_element_type=jnp.float32)
    e1p_ref[0] = e1[:, :CH]
    e1p_ref[1] = e1[:, CH:]
    e2p_ref[0] = e2[:, :CH]
    e2p_ref[1] = e2[:, CH:]


def _edges(edge_attr, c1p, c2p):
    nblk = E // BLK_E
    return pl.pallas_call(
        _edges_body,
        grid=(nblk,),
        in_specs=[
            pl.BlockSpec((BLK_E, D_EDGE), lambda i: (i, 0)),
            pl.BlockSpec((HID, 2 * HID), lambda i: (0, 0)),
            pl.BlockSpec((HID, D_EDGE), lambda i: (0, 0)),
            pl.BlockSpec((HID, 2 * HID), lambda i: (0, 0)),
            pl.BlockSpec((HID, D_EDGE), lambda i: (0, 0)),
        ],
        out_specs=(
            pl.BlockSpec((NC, BLK_E, CH), lambda i: (0, i, 0)),
            pl.BlockSpec((NC, BLK_E, CH), lambda i: (0, i, 0)),
        ),
        out_shape=(
            jax.ShapeDtypeStruct((NC, E, CH), jnp.float32),
            jax.ShapeDtypeStruct((NC, E, CH), jnp.float32),
        ),
    )(edge_attr, c1p['Wm'], c1p['We'], c2p['Wm'], c2p['We'])


def _sc_conv_body(z_hbm, e_hbm, src_hbm, dst_hbm, out_hbm,
                  srcv, dstv, gv, ev, mv, aggr, sem):
    c = lax.axis_index("c")
    s = lax.axis_index("s")

    @pl.loop(0, CHUNK)
    def _zero(i):
        for j in range(CH // LANES):
            mv.at[i, pl.ds(j * LANES, LANES)][...] = jnp.zeros(
                (LANES,), jnp.float32)

    @pl.loop(0, ROW_ITERS)
    def _zcopy(it):
        blk = it * NS + s

        @pl.when(blk < NROWBLK)
        def _():
            pltpu.sync_copy(mv, aggr.at[pl.ds(blk * CHUNK, CHUNK)])

    @pl.when(s == 0)
    def _ztail():
        pltpu.sync_copy(mv.at[pl.ds(0, NROWTAIL)],
                        aggr.at[pl.ds(NROWBLK * CHUNK, NROWTAIL)])

    plsc.subcore_barrier()

    @pl.loop(0, SC_ITERS)
    def _chunk(it):
        k_id = it * NS + s

        @pl.when(k_id < NCHUNK)
        def _():
            base = k_id * CHUNK
            pltpu.sync_copy(src_hbm.at[pl.ds(base, CHUNK)], srcv)
            pltpu.sync_copy(dst_hbm.at[pl.ds(base, CHUNK)], dstv)
            pltpu.async_copy(z_hbm.at[c].at[srcv], gv, sem).wait()
            pltpu.sync_copy(e_hbm.at[c].at[pl.ds(base, CHUNK)], ev)

            @pl.loop(0, CHUNK)
            def _edge(i):
                for j in range(CH // LANES):
                    sl = pl.ds(j * LANES, LANES)
                    mv.at[i, sl][...] = jnp.maximum(
                        gv.at[i, sl][...] + ev.at[i, sl][...], 0.0)

            pltpu.sync_copy(mv, aggr.at[dstv], add=True)

    plsc.subcore_barrier()

    @pl.loop(0, ROW_ITERS)
    def _dump(it):
        blk = it * NS + s

        @pl.when(blk < NROWBLK)
        def _():
            pltpu.sync_copy(aggr.at[pl.ds(blk * CHUNK, CHUNK)],
                            out_hbm.at[c].at[pl.ds(blk * CHUNK, CHUNK)])

    @pl.when(s == 0)
    def _dtail():
        pltpu.sync_copy(aggr.at[pl.ds(NROWBLK * CHUNK, NROWTAIL)],
                        out_hbm.at[c].at[pl.ds(NROWBLK * CHUNK, NROWTAIL)])


def _sc_conv(zp, ep, src, dst):
    mesh = plsc.VectorSubcoreMesh(core_axis_name="c", subcore_axis_name="s")
    return pl.kernel(
        _sc_conv_body,
        out_type=jax.ShapeDtypeStruct((NC, N, CH), jnp.float32),
        mesh=mesh,
        scratch_types=[
            pltpu.VMEM((CHUNK,), jnp.int32),
            pltpu.VMEM((CHUNK,), jnp.int32),
            pltpu.VMEM((CHUNK, CH), jnp.float32),
            pltpu.VMEM((CHUNK, CH), jnp.float32),
            pltpu.VMEM((CHUNK, CH), jnp.float32),
            pltpu.VMEM_SHARED((N, CH), jnp.float32),
            pltpu.SemaphoreType.DMA,
        ],
    )(zp, ep, src, dst)


def _update_body(old_ref, aggr_ref, wg_ref, bg_ref, wup_ref, bup_ref,
                 hpre_ref, stats_ref):
    i = pl.program_id(0)
    old = old_ref[...]
    cat = jnp.concatenate([old, aggr_ref[0], aggr_ref[1]], axis=-1)
    gate = jax.nn.sigmoid(jnp.dot(cat, wg_ref[...].T,
                                  preferred_element_type=jnp.float32)
                          + bg_ref[...])
    upd = jax.nn.relu(jnp.dot(cat, wup_ref[...].T,
                              preferred_element_type=jnp.float32)
                      + bup_ref[...])
    h = gate * upd + (1.0 - gate) * old
    hpre_ref[...] = h

    @pl.when(i == 0)
    def _():
        stats_ref[...] = jnp.zeros_like(stats_ref)

    stats_ref[0:1, :] += jnp.sum(h, axis=0, keepdims=True)
    stats_ref[1:2, :] += jnp.sum(h * h, axis=0, keepdims=True)


def _update(old, aggrp, cp):
    nblk = N // BLK_N
    return pl.pallas_call(
        _update_body,
        grid=(nblk,),
        in_specs=[
            pl.BlockSpec((BLK_N, HID), lambda i: (i, 0)),
            pl.BlockSpec((NC, BLK_N, CH), lambda i: (0, i, 0)),
            pl.BlockSpec((HID, 2 * HID), lambda i: (0, 0)),
            pl.BlockSpec((1, HID), lambda i: (0, 0)),
            pl.BlockSpec((HID, 2 * HID), lambda i: (0, 0)),
            pl.BlockSpec((1, HID), lambda i: (0, 0)),
        ],
        out_specs=(
            pl.BlockSpec((BLK_N, HID), lambda i: (i, 0)),
            pl.BlockSpec((8, HID), lambda i: (0, 0)),
        ),
        out_shape=(
            jax.ShapeDtypeStruct((N, HID), jnp.float32),
            jax.ShapeDtypeStruct((8, HID), jnp.float32),
        ),
    )(old, aggrp, cp['Wg'], cp['bg'].reshape(1, HID),
      cp['Wup'], cp['bup'].reshape(1, HID))


def _bn_next_body(hpre_ref, stats_ref, gamma_ref, beta_ref,
                  a2_ref, c2_ref, wux_ref, bux_ref,
                  z2p_ref, old2_ref):
    mean = stats_ref[0:1, :] / N
    var = stats_ref[1:2, :] / N - mean * mean
    h = jax.nn.relu(gamma_ref[...] * (hpre_ref[...] - mean)
                    * jax.lax.rsqrt(var + 1e-5) + beta_ref[...])
    z2 = jnp.dot(h, a2_ref[...].T, preferred_element_type=jnp.float32) \
        + c2_ref[...]
    z2p_ref[0] = z2[:, :CH]
    z2p_ref[1] = z2[:, CH:]
    old2_ref[...] = jnp.dot(h, wux_ref[...].T,
                            preferred_element_type=jnp.float32) + bux_ref[...]


def _bn_next(hpre, stats, gamma, beta, a2, c2, c2p):
    nblk = N // BLK_N
    return pl.pallas_call(
        _bn_next_body,
        grid=(nblk,),
        in_specs=[
            pl.BlockSpec((BLK_N, HID), lambda i: (i, 0)),
            pl.BlockSpec((8, HID), lambda i: (0, 0)),
            pl.BlockSpec((1, HID), lambda i: (0, 0)),
            pl.BlockSpec((1, HID), lambda i: (0, 0)),
            pl.BlockSpec((HID, HID), lambda i: (0, 0)),
            pl.BlockSpec((1, HID), lambda i: (0, 0)),
            pl.BlockSpec((HID, HID), lambda i: (0, 0)),
            pl.BlockSpec((1, HID), lambda i: (0, 0)),
        ],
        out_specs=(
            pl.BlockSpec((NC, BLK_N, CH), lambda i: (0, i, 0)),
            pl.BlockSpec((BLK_N, HID), lambda i: (i, 0)),
        ),
        out_shape=(
            jax.ShapeDtypeStruct((NC, N, CH), jnp.float32),
            jax.ShapeDtypeStruct((N, HID), jnp.float32),
        ),
    )(hpre, stats, gamma.reshape(1, HID), beta.reshape(1, HID),
      a2, c2, c2p['Wux'], c2p['bux'].reshape(1, HID))


def _bn_body(hpre_ref, stats_ref, gamma_ref, beta_ref, h_ref):
    mean = stats_ref[0:1, :] / N
    var = stats_ref[1:2, :] / N - mean * mean
    h_ref[...] = jax.nn.relu(gamma_ref[...] * (hpre_ref[...] - mean)
                             * jax.lax.rsqrt(var + 1e-5) + beta_ref[...])


def _bn(hpre, stats, gamma, beta):
    nblk = N // BLK_N
    return pl.pallas_call(
        _bn_body,
        grid=(nblk,),
        in_specs=[
            pl.BlockSpec((BLK_N, HID), lambda i: (i, 0)),
            pl.BlockSpec((8, HID), lambda i: (0, 0)),
            pl.BlockSpec((1, HID), lambda i: (0, 0)),
            pl.BlockSpec((1, HID), lambda i: (0, 0)),
        ],
        out_specs=pl.BlockSpec((BLK_N, HID), lambda i: (i, 0)),
        out_shape=jax.ShapeDtypeStruct((N, HID), jnp.float32),
    )(hpre, stats, gamma.reshape(1, HID), beta.reshape(1, HID))


def _head_body(h_ref, batch_ref, wfc_ref, bfc_ref, whead_ref, bhead_ref,
               out_ref):
    gid = lax.broadcasted_iota(jnp.int32, (G, N), 0)
    onehot = (batch_ref[...] == gid).astype(jnp.float32)
    sums = jnp.dot(onehot, h_ref[...], preferred_element_type=jnp.float32)
    counts = jnp.sum(onehot, axis=1, keepdims=True)
    gf = sums / jnp.maximum(counts, 1.0)
    gf = jax.nn.relu(jnp.dot(gf, wfc_ref[...].T,
                             preferred_element_type=jnp.float32) + bfc_ref[...])
    res = jnp.dot(gf, whead_ref[...].T,
                  preferred_element_type=jnp.float32) + bhead_ref[...]
    col = lax.broadcasted_iota(jnp.int32, (G, 2), 1)
    out_ref[...] = jnp.where(col == 0, res, jax.nn.sigmoid(res))


def _head(h, batch, params):
    whead = jnp.concatenate([params['Wreg'], params['Wcls']], axis=0)
    bhead = jnp.concatenate([params['breg'], params['bcls']]).reshape(1, 2)
    return pl.pallas_call(
        _head_body,
        out_shape=jax.ShapeDtypeStruct((G, 2), jnp.float32),
    )(h, batch.reshape(1, N), params['Wfc'],
      params['bfc'].reshape(1, HID // 2), whead, bhead)


def kernel(x, edge_index, edge_attr, batch, params):
    src = edge_index[0]
    dst = edge_index[1]
    c1p = params['conv1']
    c2p = params['conv2']

    z1p, old1, a2, c2 = _prep(x, params['feature_gates'].reshape(1, D_IN),
                              params['Win'], params['bin'].reshape(1, BOT),
                              c1p, c2p)
    e1p, e2p = _edges(edge_attr, c1p, c2p)

    aggr1p = _sc_conv(z1p, e1p, src, dst)
    hpre1, stats1 = _update(old1, aggr1p, c1p)
    z2p, old2 = _bn_next(hpre1, stats1, params['g1'], params['b1'],
                         a2, c2, c2p)

    aggr2p = _sc_conv(z2p, e2p, src, dst)
    hpre2, stats2 = _update(old2, aggr2p, c2p)
    h2 = _bn(hpre2, stats2, params['g2'], params['b2'])

    out = _head(h2, batch, params)
    return (out[:, 0], out[:, 1])


# speedup vs baseline: 4.2006x; 1.8251x over previous
import jax
import jax.numpy as jnp
from jax import lax
from jax.experimental import pallas as pl
from jax.experimental.pallas import tpu as pltpu
from jax.experimental.pallas import tpu_sc as plsc

N = 10000
E = 320000
D_IN = 128
D_EDGE = 16
BOT = 8
HID = 256
G = 64

NC = 2
NS = 16
LANES = 16
CH = HID // NC
CHUNK = 64
NCHUNK = E // CHUNK
ROWBLK = 128
NROWBLK = N // ROWBLK
NROWTAIL = N - NROWBLK * ROWBLK
ROW_ITERS = (NROWBLK + NS - 1) // NS
SC_ITERS = (NCHUNK + NS - 1) // NS

BLK_E = 4000
BLK_N = 2000


def _prep_body(x_ref, gates_ref, win_ref, bin_ref,
               ws1_ref, bs1_ref, wm1_ref, bm1_ref, wux1_ref, bux1_ref,
               z1p_ref, old1_ref):
    h = x_ref[...] * jax.nn.sigmoid(gates_ref[...])
    h = jax.nn.relu(jnp.dot(h, win_ref[...].T,
                            preferred_element_type=jnp.float32) + bin_ref[...])
    np1 = jnp.dot(h, ws1_ref[...].T,
                  preferred_element_type=jnp.float32) + bs1_ref[...]
    z1 = jnp.dot(np1, wm1_ref[:, :HID].T,
                 preferred_element_type=jnp.float32) + bm1_ref[...]
    z1p_ref[0] = z1[:, :CH]
    z1p_ref[1] = z1[:, CH:]
    old1_ref[...] = jnp.dot(h, wux1_ref[...].T,
                            preferred_element_type=jnp.float32) + bux1_ref[...]


def _prep(x, gates, win, bin_, c1p):
    return pl.pallas_call(
        _prep_body,
        out_shape=(
            jax.ShapeDtypeStruct((NC, N, CH), jnp.float32),
            jax.ShapeDtypeStruct((N, HID), jnp.float32),
        ),
    )(x, gates, win, bin_,
      c1p['Ws'], c1p['bs'].reshape(1, HID), c1p['Wm'],
      c1p['bm'].reshape(1, HID),
      c1p['Wux'], c1p['bux'].reshape(1, HID))


def _edges_body(ea_ref, wm1_ref, we1_ref, be1_ref, wm2_ref, we2_ref, be2_ref,
                e1p_ref, e2p_ref):
    ea = ea_ref[...]
    ep1 = jnp.dot(ea, we1_ref[...].T,
                  preferred_element_type=jnp.float32) + be1_ref[...]
    e1 = jnp.dot(ep1, wm1_ref[:, HID:].T, preferred_element_type=jnp.float32)
    ep2 = jnp.dot(ea, we2_ref[...].T,
                  preferred_element_type=jnp.float32) + be2_ref[...]
    e2 = jnp.dot(ep2, wm2_ref[:, HID:].T, preferred_element_type=jnp.float32)
    e1p_ref[0] = e1[:, :CH]
    e1p_ref[1] = e1[:, CH:]
    e2p_ref[0] = e2[:, :CH]
    e2p_ref[1] = e2[:, CH:]


def _edges(edge_attr, c1p, c2p):
    nblk = E // BLK_E
    return pl.pallas_call(
        _edges_body,
        grid=(nblk,),
        in_specs=[
            pl.BlockSpec((BLK_E, D_EDGE), lambda i: (i, 0)),
            pl.BlockSpec((HID, 2 * HID), lambda i: (0, 0)),
            pl.BlockSpec((HID, D_EDGE), lambda i: (0, 0)),
            pl.BlockSpec((1, HID), lambda i: (0, 0)),
            pl.BlockSpec((HID, 2 * HID), lambda i: (0, 0)),
            pl.BlockSpec((HID, D_EDGE), lambda i: (0, 0)),
            pl.BlockSpec((1, HID), lambda i: (0, 0)),
        ],
        out_specs=(
            pl.BlockSpec((NC, BLK_E, CH), lambda i: (0, i, 0)),
            pl.BlockSpec((NC, BLK_E, CH), lambda i: (0, i, 0)),
        ),
        out_shape=(
            jax.ShapeDtypeStruct((NC, E, CH), jnp.float32),
            jax.ShapeDtypeStruct((NC, E, CH), jnp.float32),
        ),
    )(edge_attr, c1p['Wm'], c1p['We'], c1p['be'].reshape(1, HID),
      c2p['Wm'], c2p['We'], c2p['be'].reshape(1, HID))


def _sc_conv_body(z_hbm, e_hbm, src_hbm, dst_hbm, out_hbm,
                  src0, src1, dst0, dst1, ds0, ds1, g0, g1, e0, e1, m0, m1,
                  aggr,
                  semi0, semi1, semg0, semg1, seme0, seme1, sems0, sems1):
    c = lax.axis_index("c")
    s = lax.axis_index("s")

    srcv = (src0, src1)
    dstv = (dst0, dst1)
    dscat = (ds0, ds1)
    gbuf = (g0, g1)
    ebuf = (e0, e1)
    mbuf = (m0, m1)
    semi = (semi0, semi1)
    semg = (semg0, semg1)
    seme = (seme0, seme1)
    sems = (sems0, sems1)

    for m in mbuf:
        @pl.loop(0, CHUNK)
        def _zero(i):
            for j in range(CH // LANES):
                m.at[i, pl.ds(j * LANES, LANES)][...] = jnp.zeros(
                    (LANES,), jnp.float32)

    @pl.loop(0, ROW_ITERS)
    def _zcopy(it):
        blk = it * NS + s

        @pl.when(blk < NROWBLK)
        def _():
            pltpu.sync_copy(m0, aggr.at[pl.ds(blk * ROWBLK, CHUNK)])
            pltpu.sync_copy(m1, aggr.at[pl.ds(blk * ROWBLK + CHUNK, CHUNK)])

    @pl.when(s == 0)
    def _ztail():
        pltpu.sync_copy(m0.at[pl.ds(0, NROWTAIL)],
                        aggr.at[pl.ds(NROWBLK * ROWBLK, NROWTAIL)])

    plsc.subcore_barrier()

    def _chunk_id(it):
        return it * NS + s

    def _issue_idx(it, b):
        @pl.when(_chunk_id(it) < NCHUNK)
        def _():
            base = _chunk_id(it) * CHUNK
            pltpu.async_copy(src_hbm.at[pl.ds(base, CHUNK)], srcv[b], semi[b])
            pltpu.async_copy(dst_hbm.at[pl.ds(base, CHUNK)], dstv[b], semi[b])

    def _wait_idx(it, b):
        @pl.when(_chunk_id(it) < NCHUNK)
        def _():
            pltpu.make_async_copy(src_hbm.at[pl.ds(0, CHUNK)], srcv[b],
                                  semi[b]).wait()
            pltpu.make_async_copy(dst_hbm.at[pl.ds(0, CHUNK)], dstv[b],
                                  semi[b]).wait()

    def _issue_data(it, b):
        @pl.when(_chunk_id(it) < NCHUNK)
        def _():
            base = _chunk_id(it) * CHUNK
            pltpu.async_copy(z_hbm.at[c].at[srcv[b]], gbuf[b], semg[b])
            pltpu.async_copy(e_hbm.at[c].at[pl.ds(base, CHUNK)], ebuf[b],
                             seme[b])

    _issue_idx(0, 0)
    _wait_idx(0, 0)
    _issue_data(0, 0)
    _issue_idx(1, 1)

    @pl.loop(0, SC_ITERS, step=2)
    def _chunk(it0):
        for half in range(2):
            it = it0 + half
            b = half
            nb = 1 - half
            orig = _chunk_id(it)

            _wait_idx(it + 1, nb)
            _issue_data(it + 1, nb)

            @pl.when(orig < NCHUNK)
            def _():
                pltpu.make_async_copy(z_hbm.at[c].at[srcv[b]], gbuf[b],
                                      semg[b]).wait()
                pltpu.make_async_copy(e_hbm.at[c].at[pl.ds(0, CHUNK)],
                                      ebuf[b], seme[b]).wait()

                @pl.when(it >= 2)
                def _():
                    pltpu.make_async_copy(mbuf[b], aggr.at[dscat[b]],
                                          sems[b]).wait()

                for j in range(CHUNK // LANES):
                    sl = pl.ds(j * LANES, LANES)
                    dscat[b].at[sl][...] = dstv[b].at[sl][...]

            _issue_idx(it + 2, b)

            @pl.when(orig < NCHUNK)
            def _():
                @pl.loop(0, CHUNK)
                def _edge(i):
                    for j in range(CH // LANES):
                        sl = pl.ds(j * LANES, LANES)
                        mbuf[b].at[i, sl][...] = jnp.maximum(
                            gbuf[b].at[i, sl][...] + ebuf[b].at[i, sl][...],
                            0.0)

                pltpu.async_copy(mbuf[b], aggr.at[dscat[b]], sems[b],
                                 add=True)

    for b in range(2):
        pltpu.make_async_copy(mbuf[b], aggr.at[dscat[b]], sems[b]).wait()

    plsc.subcore_barrier()

    @pl.loop(0, ROW_ITERS)
    def _dump(it):
        blk = it * NS + s

        @pl.when(blk < NROWBLK)
        def _():
            pltpu.sync_copy(aggr.at[pl.ds(blk * ROWBLK, ROWBLK)],
                            out_hbm.at[c].at[pl.ds(blk * ROWBLK, ROWBLK)])

    @pl.when(s == 0)
    def _dtail():
        pltpu.sync_copy(aggr.at[pl.ds(NROWBLK * ROWBLK, NROWTAIL)],
                        out_hbm.at[c].at[pl.ds(NROWBLK * ROWBLK, NROWTAIL)])


def _sc_conv(zp, ep, src, dst):
    mesh = plsc.VectorSubcoreMesh(core_axis_name="c", subcore_axis_name="s")
    return pl.kernel(
        _sc_conv_body,
        out_type=jax.ShapeDtypeStruct((NC, N, CH), jnp.float32),
        mesh=mesh,
        scratch_types=[
            pltpu.VMEM((CHUNK,), jnp.int32),
            pltpu.VMEM((CHUNK,), jnp.int32),
            pltpu.VMEM((CHUNK,), jnp.int32),
            pltpu.VMEM((CHUNK,), jnp.int32),
            pltpu.VMEM((CHUNK,), jnp.int32),
            pltpu.VMEM((CHUNK,), jnp.int32),
            pltpu.VMEM((CHUNK, CH), jnp.float32),
            pltpu.VMEM((CHUNK, CH), jnp.float32),
            pltpu.VMEM((CHUNK, CH), jnp.float32),
            pltpu.VMEM((CHUNK, CH), jnp.float32),
            pltpu.VMEM((CHUNK, CH), jnp.float32),
            pltpu.VMEM((CHUNK, CH), jnp.float32),
            pltpu.VMEM_SHARED((N, CH), jnp.float32),
            pltpu.SemaphoreType.DMA,
            pltpu.SemaphoreType.DMA,
            pltpu.SemaphoreType.DMA,
            pltpu.SemaphoreType.DMA,
            pltpu.SemaphoreType.DMA,
            pltpu.SemaphoreType.DMA,
            pltpu.SemaphoreType.DMA,
            pltpu.SemaphoreType.DMA,
        ],
    )(zp, ep, src, dst)


def _update_body(old_ref, aggr_ref, wg_ref, bg_ref, wup_ref, bup_ref,
                 hpre_ref, stats_ref):
    i = pl.program_id(0)
    old = old_ref[...]
    cat = jnp.concatenate([old, aggr_ref[0], aggr_ref[1]], axis=-1)
    gate = jax.nn.sigmoid(jnp.dot(cat, wg_ref[...].T,
                                  preferred_element_type=jnp.float32)
                          + bg_ref[...])
    upd = jax.nn.relu(jnp.dot(cat, wup_ref[...].T,
                              preferred_element_type=jnp.float32)
                      + bup_ref[...])
    h = gate * upd + (1.0 - gate) * old
    hpre_ref[...] = h

    @pl.when(i == 0)
    def _():
        stats_ref[...] = jnp.zeros_like(stats_ref)

    stats_ref[0:1, :] += jnp.sum(h, axis=0, keepdims=True)
    stats_ref[1:2, :] += jnp.sum(h * h, axis=0, keepdims=True)


def _update(old, aggrp, cp):
    nblk = N // BLK_N
    return pl.pallas_call(
        _update_body,
        grid=(nblk,),
        in_specs=[
            pl.BlockSpec((BLK_N, HID), lambda i: (i, 0)),
            pl.BlockSpec((NC, BLK_N, CH), lambda i: (0, i, 0)),
            pl.BlockSpec((HID, 2 * HID), lambda i: (0, 0)),
            pl.BlockSpec((1, HID), lambda i: (0, 0)),
            pl.BlockSpec((HID, 2 * HID), lambda i: (0, 0)),
            pl.BlockSpec((1, HID), lambda i: (0, 0)),
        ],
        out_specs=(
            pl.BlockSpec((BLK_N, HID), lambda i: (i, 0)),
            pl.BlockSpec((8, HID), lambda i: (0, 0)),
        ),
        out_shape=(
            jax.ShapeDtypeStruct((N, HID), jnp.float32),
            jax.ShapeDtypeStruct((8, HID), jnp.float32),
        ),
    )(old, aggrp, cp['Wg'], cp['bg'].reshape(1, HID),
      cp['Wup'], cp['bup'].reshape(1, HID))


def _bn_next_body(hpre_ref, stats_ref, gamma_ref, beta_ref,
                  ws2_ref, bs2_ref, wm2_ref, bm2_ref, wux_ref, bux_ref,
                  z2p_ref, old2_ref):
    mean = stats_ref[0:1, :] / N
    var = stats_ref[1:2, :] / N - mean * mean
    h = jax.nn.relu(gamma_ref[...] * (hpre_ref[...] - mean)
                    * jax.lax.rsqrt(var + 1e-5) + beta_ref[...])
    np2 = jnp.dot(h, ws2_ref[...].T,
                  preferred_element_type=jnp.float32) + bs2_ref[...]
    z2 = jnp.dot(np2, wm2_ref[:, :HID].T,
                 preferred_element_type=jnp.float32) + bm2_ref[...]
    z2p_ref[0] = z2[:, :CH]
    z2p_ref[1] = z2[:, CH:]
    old2_ref[...] = jnp.dot(h, wux_ref[...].T,
                            preferred_element_type=jnp.float32) + bux_ref[...]


def _bn_next(hpre, stats, gamma, beta, c2p):
    nblk = N // BLK_N
    return pl.pallas_call(
        _bn_next_body,
        grid=(nblk,),
        in_specs=[
            pl.BlockSpec((BLK_N, HID), lambda i: (i, 0)),
            pl.BlockSpec((8, HID), lambda i: (0, 0)),
            pl.BlockSpec((1, HID), lambda i: (0, 0)),
            pl.BlockSpec((1, HID), lambda i: (0, 0)),
            pl.BlockSpec((HID, HID), lambda i: (0, 0)),
            pl.BlockSpec((1, HID), lambda i: (0, 0)),
            pl.BlockSpec((HID, 2 * HID), lambda i: (0, 0)),
            pl.BlockSpec((1, HID), lambda i: (0, 0)),
            pl.BlockSpec((HID, HID), lambda i: (0, 0)),
            pl.BlockSpec((1, HID), lambda i: (0, 0)),
        ],
        out_specs=(
            pl.BlockSpec((NC, BLK_N, CH), lambda i: (0, i, 0)),
            pl.BlockSpec((BLK_N, HID), lambda i: (i, 0)),
        ),
        out_shape=(
            jax.ShapeDtypeStruct((NC, N, CH), jnp.float32),
            jax.ShapeDtypeStruct((N, HID), jnp.float32),
        ),
    )(hpre, stats, gamma.reshape(1, HID), beta.reshape(1, HID),
      c2p['Ws'], c2p['bs'].reshape(1, HID), c2p['Wm'],
      c2p['bm'].reshape(1, HID), c2p['Wux'], c2p['bux'].reshape(1, HID))


def _bn_body(hpre_ref, stats_ref, gamma_ref, beta_ref, h_ref):
    mean = stats_ref[0:1, :] / N
    var = stats_ref[1:2, :] / N - mean * mean
    h_ref[...] = jax.nn.relu(gamma_ref[...] * (hpre_ref[...] - mean)
                             * jax.lax.rsqrt(var + 1e-5) + beta_ref[...])


def _bn(hpre, stats, gamma, beta):
    nblk = N // BLK_N
    return pl.pallas_call(
        _bn_body,
        grid=(nblk,),
        in_specs=[
            pl.BlockSpec((BLK_N, HID), lambda i: (i, 0)),
            pl.BlockSpec((8, HID), lambda i: (0, 0)),
            pl.BlockSpec((1, HID), lambda i: (0, 0)),
            pl.BlockSpec((1, HID), lambda i: (0, 0)),
        ],
        out_specs=pl.BlockSpec((BLK_N, HID), lambda i: (i, 0)),
        out_shape=jax.ShapeDtypeStruct((N, HID), jnp.float32),
    )(hpre, stats, gamma.reshape(1, HID), beta.reshape(1, HID))


def _head_body(h_ref, batch_ref, wfc_ref, bfc_ref, whead_ref, bhead_ref,
               out_ref):
    gid = lax.broadcasted_iota(jnp.int32, (G, N), 0)
    onehot = (batch_ref[...] == gid).astype(jnp.float32)
    sums = jnp.dot(onehot, h_ref[...], precision=lax.Precision.HIGHEST,
                   preferred_element_type=jnp.float32)
    counts = jnp.sum(onehot, axis=1, keepdims=True)
    gf = sums / jnp.maximum(counts, 1.0)
    gf = jax.nn.relu(jnp.dot(gf, wfc_ref[...].T,
                             preferred_element_type=jnp.float32) + bfc_ref[...])
    res = jnp.dot(gf, whead_ref[...].T,
                  preferred_element_type=jnp.float32) + bhead_ref[...]
    col = lax.broadcasted_iota(jnp.int32, (G, 2), 1)
    out_ref[...] = jnp.where(col == 0, res, jax.nn.sigmoid(res))


def _head(h, batch, params):
    whead = jnp.concatenate([params['Wreg'], params['Wcls']], axis=0)
    bhead = jnp.concatenate([params['breg'], params['bcls']]).reshape(1, 2)
    return pl.pallas_call(
        _head_body,
        out_shape=jax.ShapeDtypeStruct((G, 2), jnp.float32),
    )(h, batch.reshape(1, N), params['Wfc'],
      params['bfc'].reshape(1, HID // 2), whead, bhead)


def kernel(x, edge_index, edge_attr, batch, params):
    src = edge_index[0]
    dst = edge_index[1]
    c1p = params['conv1']
    c2p = params['conv2']

    z1p, old1 = _prep(x, params['feature_gates'].reshape(1, D_IN),
                      params['Win'], params['bin'].reshape(1, BOT), c1p)
    e1p, e2p = _edges(edge_attr, c1p, c2p)

    aggr1p = _sc_conv(z1p, e1p, src, dst)
    hpre1, stats1 = _update(old1, aggr1p, c1p)
    z2p, old2 = _bn_next(hpre1, stats1, params['g1'], params['b1'], c2p)

    aggr2p = _sc_conv(z2p, e2p, src, dst)
    hpre2, stats2 = _update(old2, aggr2p, c2p)
    h2 = _bn(hpre2, stats2, params['g2'], params['b2'])

    out = _head(h2, batch, params)
    return (out[:, 0], out[:, 1])
